# Initial kernel scaffold; baseline (speedup 1.0000x reference)
import jax
import jax.numpy as jnp
from jax.experimental import pallas as pl


def kernel(x, codebook):
    raise NotImplementedError("write your pallas kernel here")


# speedup vs baseline: 1.1921x; 1.1921x over previous
import functools

import jax
import jax.numpy as jnp
from jax import lax
from jax.experimental import pallas as pl
from jax.experimental.pallas import tpu as pltpu
from jax.experimental.pallas import tpu_sc as plsc

B, D, K = 16384, 64, 8192
BM = 256
G = B // BM


DK = 256


def _argmin_body(x_ref, cbt_ref, cnorm_ref, xn_ref, ids_ref):
    x_blk = x_ref[...]
    mm = lax.dot_general(
        x_blk, cbt_ref[...],
        (((1,), (0,)), ((), ())),
        preferred_element_type=jnp.float32,
    )
    xn = xn_ref[:, 0:1]
    dist = (xn - 2.0 * mm) + cnorm_ref[...]
    minval = jnp.min(dist, axis=1, keepdims=True)
    iota = lax.broadcasted_iota(jnp.int32, dist.shape, 1)
    cand = jnp.where(dist == minval, iota, jnp.int32(K))
    ids_row = jnp.min(cand, axis=1)
    ids_ref[0] = jnp.broadcast_to(ids_row[None, :], (8, BM))


def _argmin_ids(xp, cbtp, cnorm, xnb):
    out = pl.pallas_call(
        _argmin_body,
        grid=(G,),
        in_specs=[
            pl.BlockSpec((BM, DK), lambda i: (i, 0)),
            pl.BlockSpec((DK, K), lambda i: (0, 0)),
            pl.BlockSpec((1, K), lambda i: (0, 0)),
            pl.BlockSpec((BM, 128), lambda i: (i, 0)),
        ],
        out_specs=pl.BlockSpec((1, 8, BM), lambda i: (i, 0, 0)),
        out_shape=jax.ShapeDtypeStruct((G, 8, BM), jnp.int32),
        compiler_params=pltpu.CompilerParams(
            dimension_semantics=("arbitrary",),
        ),
    )(xp, cbtp, cnorm, xnb)
    return out[:, 0, :].reshape(B)


DP = 128


def _make_sc_gather():
    info = plsc.get_sparse_core_info()
    nc, ns = info.num_cores, info.num_subcores
    nw = nc * ns
    b_per_w = B // nw

    mesh = plsc.VectorSubcoreMesh(core_axis_name="c", subcore_axis_name="s")

    @functools.partial(
        pl.kernel, mesh=mesh,
        out_type=jax.ShapeDtypeStruct((B, DP), jnp.float32),
        scratch_types=[
            pltpu.VMEM((b_per_w,), jnp.int32),
            pltpu.VMEM((b_per_w, DP), jnp.float32),
            pltpu.SemaphoreType.DMA,
        ],
    )
    def gather_rows(table_hbm, idx_hbm, out_hbm, idx_v, rows_v, sem):
        wid = lax.axis_index("s") * nc + lax.axis_index("c")
        base = wid * b_per_w
        pltpu.sync_copy(idx_hbm.at[pl.ds(base, b_per_w)], idx_v)
        pltpu.async_copy(table_hbm.at[idx_v], rows_v, sem).wait()
        pltpu.sync_copy(rows_v, out_hbm.at[pl.ds(base, b_per_w)])

    return gather_rows


_sc_gather = _make_sc_gather()


def kernel(x, codebook):
    xp = jnp.concatenate([x, jnp.zeros((B, DK - D), jnp.float32)], axis=1)
    cbtp = jnp.concatenate(
        [codebook.T, jnp.zeros((DK - D, K), jnp.float32)], axis=0)
    cnorm = jnp.sum(codebook ** 2, axis=1)[None, :]
    xnorm = jnp.sum(x ** 2, axis=1, keepdims=True)
    xnb = jnp.broadcast_to(xnorm, (B, 128))
    ids = _argmin_ids(xp, cbtp, cnorm, xnb)
    cb_pad = jnp.concatenate(
        [codebook, jnp.zeros((K, DP - D), jnp.float32)], axis=1)
    e_k = _sc_gather(cb_pad, ids)[:, :D]
    e_k_st = x + (e_k - x)
    return (e_k_st, ids)


# speedup vs baseline: 1.1927x; 1.0005x over previous
import functools

import jax
import jax.numpy as jnp
from jax import lax
from jax.experimental import pallas as pl
from jax.experimental.pallas import tpu as pltpu
from jax.experimental.pallas import tpu_sc as plsc

B, D, K = 16384, 64, 8192
BM = 256
G = B // BM


DK = 256


def _argmin_body(x_ref, cbt_ref, cnorm_ref, xn_ref, ids_ref):
    x_blk = x_ref[...]
    mm = lax.dot_general(
        x_blk, cbt_ref[...],
        (((1,), (0,)), ((), ())),
        preferred_element_type=jnp.float32,
    )
    xn = xn_ref[:, 0:1]
    dist = (xn - 2.0 * mm) + cnorm_ref[...]
    minval = jnp.min(dist, axis=1, keepdims=True)
    iota = lax.broadcasted_iota(jnp.int32, dist.shape, 1)
    cand = jnp.where(dist == minval, iota, jnp.int32(K))
    ids_row = jnp.min(cand, axis=1)
    ids_ref[0] = jnp.broadcast_to(ids_row[None, :], (8, BM))


def _argmin_ids(xp, cbtp, cnorm, xnb):
    out = pl.pallas_call(
        _argmin_body,
        grid=(G,),
        in_specs=[
            pl.BlockSpec((BM, DK), lambda i: (i, 0)),
            pl.BlockSpec((DK, K), lambda i: (0, 0)),
            pl.BlockSpec((1, K), lambda i: (0, 0)),
            pl.BlockSpec((BM, 128), lambda i: (i, 0)),
        ],
        out_specs=pl.BlockSpec((1, 8, BM), lambda i: (i, 0, 0)),
        out_shape=jax.ShapeDtypeStruct((G, 8, BM), jnp.int32),
        compiler_params=pltpu.CompilerParams(
            dimension_semantics=("arbitrary",),
        ),
    )(xp, cbtp, cnorm, xnb)
    return out[:, 0, :].reshape(B)


DP = 128


def _make_sc_gather():
    info = plsc.get_sparse_core_info()
    nc, ns = info.num_cores, info.num_subcores
    nw = nc * ns
    b_per_w = B // nw

    mesh = plsc.VectorSubcoreMesh(core_axis_name="c", subcore_axis_name="s")

    @functools.partial(
        pl.kernel, mesh=mesh,
        out_type=jax.ShapeDtypeStruct((B, DP), jnp.float32),
        scratch_types=[
            pltpu.VMEM((b_per_w,), jnp.int32),
            pltpu.VMEM((b_per_w, DP), jnp.float32),
            pltpu.SemaphoreType.DMA,
        ],
    )
    def gather_rows(table_hbm, idx_hbm, out_hbm, idx_v, rows_v, sem):
        wid = lax.axis_index("s") * nc + lax.axis_index("c")
        base = wid * b_per_w
        pltpu.sync_copy(idx_hbm.at[pl.ds(base, b_per_w)], idx_v)
        pltpu.async_copy(table_hbm.at[idx_v], rows_v, sem).wait()
        pltpu.sync_copy(rows_v, out_hbm.at[pl.ds(base, b_per_w)])

    return gather_rows


_sc_gather_cache = []


def _sc_gather(table, idx):
    if not _sc_gather_cache:
        _sc_gather_cache.append(_make_sc_gather())
    return _sc_gather_cache[0](table, idx)


def kernel(x, codebook):
    xp = jnp.concatenate([x, jnp.zeros((B, DK - D), jnp.float32)], axis=1)
    cbtp = jnp.concatenate(
        [codebook.T, jnp.zeros((DK - D, K), jnp.float32)], axis=0)
    cnorm = jnp.sum(codebook ** 2, axis=1)[None, :]
    xnorm = jnp.sum(x ** 2, axis=1, keepdims=True)
    xnb = jnp.broadcast_to(xnorm, (B, 128))
    ids = _argmin_ids(xp, cbtp, cnorm, xnb)
    cb_pad = jnp.concatenate(
        [codebook, jnp.zeros((K, DP - D), jnp.float32)], axis=1)
    e_k = _sc_gather(cb_pad, ids)[:, :D]
    e_k_st = x + (e_k - x)
    return (e_k_st, ids)
